# baseline (speedup 1.0000x reference)
import functools

import jax
import jax.numpy as jnp
from jax import lax
from jax.experimental import pallas as pl
from jax.experimental.pallas import tpu as pltpu
from jax.experimental.pallas import tpu_sc as plsc

N = 10000
E = 160000
HIDDEN = 256
HALF = 128
HEADS = 4
NSUB = 16
EB = 400
EPW = E // NSUB
NBLK = EPW // EB
NPAD = 10240
RPS = NPAD // NSUB
EB2 = 80
NBLK2 = EPW // EB2
LANES = 16
EBP = 80
NBP = EPW // EBP
GRP = EBP // LANES
NW = 2 * NSUB
EPW2 = E // NW
EBL = 160
NBP2 = 32
SPAD = NBP2 * EBL
GRL = EBL // LANES


def _proj_body(x_ref, wq_ref, bq_ref, wk_ref, bk_ref, wv_ref, bv_ref,
               q0_ref, k0_ref, v0_ref, v1_ref):
    xb = x_ref[...].astype(jnp.bfloat16)

    def proj(w_ref, b_ref):
        y = lax.dot_general(xb, w_ref[...].astype(jnp.bfloat16),
                            (((1,), (1,)), ((), ())),
                            preferred_element_type=jnp.float32)
        return y + b_ref[...]

    def pack_i32(y):
        a = lax.bitcast_convert_type(
            y[:, :HALF].astype(jnp.bfloat16), jnp.uint16).astype(jnp.uint32)
        b = lax.bitcast_convert_type(
            y[:, HALF:].astype(jnp.bfloat16), jnp.uint16).astype(jnp.uint32)
        return lax.bitcast_convert_type(a | (b << 16), jnp.int32)

    q0_ref[...] = pack_i32(proj(wq_ref, bq_ref))
    k0_ref[...] = pack_i32(proj(wk_ref, bk_ref))
    v = proj(wv_ref, bv_ref)
    v0_ref[...] = v[:, :HALF]
    v1_ref[...] = v[:, HALF:]


def _proj(x, Wq, bq, Wk, bk, Wv, bv):
    RB = 2000
    w_spec = pl.BlockSpec((HIDDEN, HIDDEN), lambda i: (0, 0))
    b_spec = pl.BlockSpec((1, HIDDEN), lambda i: (0, 0))
    half_spec = pl.BlockSpec((RB, HALF), lambda i: (i, 0))
    return pl.pallas_call(
        _proj_body,
        grid=(N // RB,),
        in_specs=[pl.BlockSpec((RB, HIDDEN), lambda i: (i, 0)),
                  w_spec, b_spec, w_spec, b_spec, w_spec, b_spec],
        out_specs=[half_spec] * 4,
        out_shape=[jax.ShapeDtypeStruct((N, HALF), jnp.int32)] * 2
        + [jax.ShapeDtypeStruct((N, HALF), jnp.float32)] * 2,
    )(x, Wq, bq.reshape(1, HIDDEN), Wk, bk.reshape(1, HIDDEN),
      Wv, bv.reshape(1, HIDDEN))


def _logits_sc(qpack, kpack, src, tgt):
    mesh = plsc.VectorSubcoreMesh(core_axis_name="c", subcore_axis_name="s")

    @functools.partial(
        pl.kernel, mesh=mesh,
        compiler_params=pltpu.CompilerParams(needs_layout_passes=False),
        out_type=tuple(jax.ShapeDtypeStruct((E,), jnp.float32)
                       for _ in range(HEADS)),
        scratch_types=[
            pltpu.VMEM((SPAD,), jnp.int32),
            pltpu.VMEM((SPAD,), jnp.int32),
            pltpu.VMEM((EBL, HALF), jnp.int32),
            pltpu.VMEM((EBL, HALF), jnp.int32),
            pltpu.VMEM((EBL, HALF), jnp.int32),
            pltpu.VMEM((EBL, HALF), jnp.int32),
            pltpu.VMEM((SPAD,), jnp.float32),
            pltpu.VMEM((SPAD,), jnp.float32),
            pltpu.VMEM((SPAD,), jnp.float32),
            pltpu.VMEM((SPAD,), jnp.float32),
            pltpu.SemaphoreType.DMA,
            pltpu.SemaphoreType.DMA,
        ],
    )
    def logits_kernel(qh, kh, srch, tgth, o0, o1, o2, o3,
                      srcv, tgtv, qA, kA, qB, kB,
                      l0v, l1v, l2v, l3v, semA, semB):
        sid = lax.axis_index("s")
        cid = lax.axis_index("c")
        wid = cid * NSUB + sid
        base0 = wid * EPW2
        lvs = (l0v, l1v, l2v, l3v)

        pltpu.sync_copy(srch.at[pl.ds(base0, EPW2)], srcv.at[pl.ds(0, EPW2)])
        pltpu.sync_copy(tgth.at[pl.ds(base0, EPW2)], tgtv.at[pl.ds(0, EPW2)])
        zero16 = jnp.zeros((LANES,), jnp.int32)
        for off in list(range(EPW2, SPAD - 16, 16)) + [SPAD - 16]:
            srcv[pl.ds(off, LANES)] = zero16
            tgtv[pl.ds(off, LANES)] = zero16
        lane = lax.iota(jnp.int32, LANES)

        def issue(b, qb, kb, sem):
            off = b * EBL
            pltpu.async_copy(qh.at[tgtv.at[pl.ds(off, EBL)]], qb, sem)
            pltpu.async_copy(kh.at[srcv.at[pl.ds(off, EBL)]], kb, sem)

        def drain(qb, kb, sem):
            pltpu.make_async_copy(qh.at[pl.ds(0, EBL)], qb, sem).wait()
            pltpu.make_async_copy(kh.at[pl.ds(0, EBL)], kb, sem).wait()

        def compute(b, qb, kb):
            off = b * EBL

            def edots(e):
                acc = [None] * HEADS
                for j in range(8):
                    qw = plsc.bitcast(qb[e, pl.ds(16 * j, 16)], jnp.bfloat16)
                    kw = plsc.bitcast(kb[e, pl.ds(16 * j, 16)], jnp.bfloat16)
                    pw = plsc.unpack(qw * kw,
                                     format=plsc.PackFormat.INTERLEAVED)
                    lo = j // 4
                    acc[lo] = pw[0] if acc[lo] is None else acc[lo] + pw[0]
                    acc[2 + lo] = (pw[1] if acc[2 + lo] is None
                                   else acc[2 + lo] + pw[1])
                return [jnp.sum(a) for a in acc]

            def grp(g, c2):
                lvecs = [jnp.zeros((LANES,), jnp.float32) for _ in range(HEADS)]
                for i in range(LANES):
                    e = g * LANES + i
                    dots = edots(e)
                    for h in range(HEADS):
                        lvecs[h] = jnp.where(lane == i, dots[h], lvecs[h])
                for h in range(HEADS):
                    lvs[h][pl.ds(off + g * LANES, LANES)] = lvecs[h]
                return c2

            lax.fori_loop(0, GRL, grp, 0)

        issue(0, qA, kA, semA)

        def pair(t, carry):
            b0 = 2 * t
            issue(b0 + 1, qB, kB, semB)
            drain(qA, kA, semA)
            compute(b0, qA, kA)

            @pl.when(t < NBP2 // 2 - 1)
            def _():
                issue(b0 + 2, qA, kA, semA)

            drain(qB, kB, semB)
            compute(b0 + 1, qB, kB)
            return carry

        lax.fori_loop(0, NBP2 // 2, pair, 0)
        for h, oh in enumerate((o0, o1, o2, o3)):
            pltpu.sync_copy(lvs[h].at[pl.ds(0, EPW2)],
                            oh.at[pl.ds(base0, EPW2)])

    return logits_kernel(qpack, kpack, src, tgt)


ER = E // 128


def _softmax_body(r0, r1, r2, r3, d_ref, g_ref, tau_ref, wd_ref, wg_ref,
                  w0, w1, w2, w3):
    tau_c = jnp.clip(tau_ref[0], 0.5, 5.0)
    scale = 1.0 / (8.0 * tau_c)
    bias = wd_ref[0] * d_ref[...] + wg_ref[0] * g_ref[...]
    for rr, ww in ((r0, w0), (r1, w1), (r2, w2), (r3, w3)):
        logit = rr[...] * scale + bias
        m = jnp.max(logit)
        p = jnp.exp(logit - m)
        s = jnp.sum(p)
        ww[...] = p / s


def _softmax(rs, dist, galign, tau, w_dist, w_galign):
    s_spec = pl.BlockSpec(memory_space=pltpu.SMEM)
    e_spec = pl.BlockSpec((ER, 128), lambda: (0, 0))
    return pl.pallas_call(
        _softmax_body,
        in_specs=[e_spec] * 6 + [s_spec, s_spec, s_spec],
        out_specs=[e_spec] * 4,
        out_shape=[jax.ShapeDtypeStruct((ER, 128), jnp.float32)] * 4,
    )(*[r.reshape(ER, 128) for r in rs],
      dist.reshape(ER, 128), galign.reshape(ER, 128),
      tau, w_dist, w_galign)


def _scatter_sc(v0, v1, src, tgt, wa, wb, wc, wd):
    mesh = plsc.VectorSubcoreMesh(core_axis_name="c", subcore_axis_name="s")

    @functools.partial(
        pl.kernel, mesh=mesh,
        compiler_params=pltpu.CompilerParams(needs_layout_passes=False),
        out_type=(jax.ShapeDtypeStruct((NPAD, HALF), jnp.float32),
                  jax.ShapeDtypeStruct((NPAD, HALF), jnp.float32)),
        scratch_types=[
            pltpu.VMEM((EBP,), jnp.int32),
            pltpu.VMEM((EBP,), jnp.int32),
            pltpu.VMEM((EBP,), jnp.int32),
            pltpu.VMEM((EBP,), jnp.int32),
            pltpu.VMEM((EBP,), jnp.int32),
            pltpu.VMEM((EBP,), jnp.int32),
            pltpu.VMEM((EBP, HALF), jnp.float32),
            pltpu.VMEM((EBP, HALF), jnp.float32),
            pltpu.VMEM((EBP, HALF), jnp.float32),
            pltpu.VMEM((EBP,), jnp.float32),
            pltpu.VMEM((EBP,), jnp.float32),
            pltpu.VMEM((EBP,), jnp.float32),
            pltpu.VMEM((EBP,), jnp.float32),
            pltpu.VMEM((EBP,), jnp.float32),
            pltpu.VMEM((EBP,), jnp.float32),
            pltpu.VMEM_SHARED((NPAD, HALF), jnp.float32),
            pltpu.SemaphoreType.DMA,
            pltpu.SemaphoreType.DMA,
            pltpu.SemaphoreType.DMA,
            pltpu.SemaphoreType.DMA,
            pltpu.SemaphoreType.DMA,
            pltpu.SemaphoreType.DMA,
            pltpu.SemaphoreType.DMA,
            pltpu.SemaphoreType.DMA,
            pltpu.SemaphoreType.DMA,
        ],
    )
    def scatter_kernel(v0h, v1h, srch, tgth, wah, wbh, wch, wdh, out0, out1,
                       iA, iB, iC, tA, tB, tC, vA, vB, vC,
                       w0A, w1A, w0B, w1B, w0C, w1C, acc,
                       gA, gB, gC, sA, sB, sC, siA, siB, siC):
        sid = lax.axis_index("s")
        cid = lax.axis_index("c")
        base0 = sid * EPW
        bufs = ((vA, tA, w0A, w1A, gA, sA),
                (vB, tB, w0B, w1B, gB, sB),
                (vC, tC, w0C, w1C, gC, sC))
        ibufs = ((iA, siA), (iB, siB), (iC, siC))

        def zrow(r, c2):
            for j in range(HALF // LANES):
                vA[r, pl.ds(16 * j, LANES)] = jnp.zeros((LANES,), jnp.float32)
            return c2

        lax.fori_loop(0, EBP, zrow, 0)
        for t in range(RPS // EBP):
            pltpu.sync_copy(vA, acc.at[pl.ds(sid * RPS + t * EBP, EBP)])
        plsc.subcore_barrier()

        def scatter_phase(vh, w0h, w1h):
            def issue_i(b, ibuf):
                ib, si = ibuf
                pltpu.async_copy(srch.at[pl.ds(base0 + b * EBP, EBP)], ib, si)

            def drain_i(ibuf):
                ib, si = ibuf
                pltpu.make_async_copy(srch.at[pl.ds(0, EBP)], ib, si).wait()

            def issue_g(b, buf, ibuf):
                vb, tb, w0b, w1b, g, _ = buf
                base = base0 + b * EBP
                pltpu.async_copy(vh.at[ibuf[0]], vb, g)
                pltpu.async_copy(w0h.at[pl.ds(base, EBP)], w0b, g)
                pltpu.async_copy(w1h.at[pl.ds(base, EBP)], w1b, g)
                pltpu.async_copy(tgth.at[pl.ds(base, EBP)], tb, g)

            def drain_g(buf):
                vb, tb, w0b, w1b, g, _ = buf
                pltpu.make_async_copy(vh.at[pl.ds(0, EBP)], vb, g).wait()
                pltpu.make_async_copy(wah.at[pl.ds(0, EBP)], w0b, g).wait()
                pltpu.make_async_copy(wah.at[pl.ds(0, EBP)], w1b, g).wait()
                pltpu.make_async_copy(srch.at[pl.ds(0, EBP)], tb, g).wait()

            def scale(buf):
                vb, _, w0b, w1b, _, _ = buf

                def grp(g, c2):
                    w0vec = w0b[pl.ds(g * LANES, LANES)]
                    w1vec = w1b[pl.ds(g * LANES, LANES)]
                    for i in range(LANES):
                        e = g * LANES + i
                        w0 = w0vec[i]
                        w1 = w1vec[i]
                        for j in range(4):
                            vb[e, pl.ds(16 * j, LANES)] = vb[e, pl.ds(16 * j, LANES)] * w0
                        for j in range(4, 8):
                            vb[e, pl.ds(16 * j, LANES)] = vb[e, pl.ds(16 * j, LANES)] * w1
                    return c2

                lax.fori_loop(0, GRP, grp, 0)

            def issue_s(buf):
                vb, tb, _, _, _, s = buf
                pltpu.async_copy(vb, acc.at[tb], s, add=True)

            def drain_s(buf):
                vb, _, _, _, _, s = buf
                pltpu.make_async_copy(vb, acc.at[pl.ds(0, EBP)], s).wait()

            pltpu.sync_copy(srch.at[pl.ds(base0, EBP)], iA)
            issue_i(1, ibufs[1])
            issue_g(0, bufs[0], ibufs[0])

            def triple(t, carry):
                b0 = 3 * t
                for r in range(3):
                    b = b0 + r
                    W = bufs[r]
                    G = bufs[(r + 1) % 3]
                    if r < 2:
                        @pl.when(t > 0)
                        def _():
                            drain_s(G)
                    else:
                        drain_s(G)
                    drain_i(ibufs[(r + 1) % 3])
                    issue_g(b + 1, G, ibufs[(r + 1) % 3])
                    drain_g(W)
                    issue_i(b + 2, ibufs[(r + 2) % 3])
                    scale(W)
                    issue_s(W)
                return carry

            lax.fori_loop(0, NBP // 3, triple, 0)
            drain_s(bufs[1])
            drain_i(ibufs[1])
            issue_g(NBP - 1, bufs[1], ibufs[1])
            drain_g(bufs[0])
            scale(bufs[0])
            issue_s(bufs[0])
            drain_s(bufs[2])
            drain_g(bufs[1])
            scale(bufs[1])
            issue_s(bufs[1])
            drain_s(bufs[0])
            drain_s(bufs[1])

        @pl.when(cid == 0)
        def _():
            scatter_phase(v0h, wah, wbh)

        @pl.when(cid == 1)
        def _():
            scatter_phase(v1h, wch, wdh)

        plsc.subcore_barrier()

        def writeback(outh):
            r = sid * RPS
            pltpu.sync_copy(acc.at[pl.ds(r, RPS)], outh.at[pl.ds(r, RPS)])

        @pl.when(cid == 0)
        def _():
            writeback(out0)

        @pl.when(cid == 1)
        def _():
            writeback(out1)

    return scatter_kernel(v0, v1, src, tgt, wa, wb, wc, wd)


def _out_body(o0_ref, o1_ref, wo_ref, bo_ref, f_ref):
    w = wo_ref[...].astype(jnp.bfloat16)
    f = lax.dot_general(o0_ref[...].astype(jnp.bfloat16), w[:, :HALF],
                        (((1,), (1,)), ((), ())),
                        preferred_element_type=jnp.float32)
    f = f + lax.dot_general(o1_ref[...].astype(jnp.bfloat16), w[:, HALF:],
                            (((1,), (1,)), ((), ())),
                            preferred_element_type=jnp.float32)
    f_ref[...] = f + bo_ref[...]


def _outproj(o0, o1, Wo, bo):
    RB = 2000
    return pl.pallas_call(
        _out_body,
        grid=(N // RB,),
        in_specs=[pl.BlockSpec((RB, HALF), lambda i: (i, 0)),
                  pl.BlockSpec((RB, HALF), lambda i: (i, 0)),
                  pl.BlockSpec((HIDDEN, HIDDEN), lambda i: (0, 0)),
                  pl.BlockSpec((1, HIDDEN), lambda i: (0, 0))],
        out_specs=pl.BlockSpec((RB, HIDDEN), lambda i: (i, 0)),
        out_shape=jax.ShapeDtypeStruct((N, HIDDEN), jnp.float32),
    )(o0, o1, Wo, bo.reshape(1, HIDDEN))


def kernel(x, edge_index, edge_attr, Wq, bq, Wk, bk, Wv, bv, Wo, bo,
           w_dist, w_galign, tau):
    src = edge_index[0].astype(jnp.int32)
    tgt = edge_index[1].astype(jnp.int32)
    dist = edge_attr[:, 2]
    galign = edge_attr[:, 3]

    qi, ki, v0, v1 = _proj(x, Wq, bq, Wk, bk, Wv, bv)
    rs = _logits_sc(qi, ki, src, tgt)
    ws = _softmax(rs, dist, galign, tau, w_dist, w_galign)
    o0p, o1p = _scatter_sc(v0, v1, src, tgt,
                           *[w.reshape(E) for w in ws])
    return _outproj(o0p, o1p, Wo, bo)


# speedup vs baseline: 1.4771x; 1.4771x over previous
import functools

import jax
import jax.numpy as jnp
from jax import lax
from jax.experimental import pallas as pl
from jax.experimental.pallas import tpu as pltpu
from jax.experimental.pallas import tpu_sc as plsc

N = 10000
E = 160000
HIDDEN = 256
HALF = 128
HEADS = 4
NSUB = 16
EB = 400
EPW = E // NSUB
NBLK = EPW // EB
NPAD = 10240
RPS = NPAD // NSUB
EB2 = 80
NBLK2 = EPW // EB2
LANES = 16
EBP = 80
NBP = EPW // EBP
GRP = EBP // LANES
NW = 2 * NSUB
EPW2 = E // NW
EBL = 80
NBP2 = 63
SPAD = NBP2 * EBL
GRL = EBL // LANES


def _proj_body(x_ref, wq_ref, bq_ref, wk_ref, bk_ref, wv_ref, bv_ref,
               q0_ref, k0_ref, v0_ref, v1_ref):
    xb = x_ref[...].astype(jnp.bfloat16)

    def proj(w_ref, b_ref):
        y = lax.dot_general(xb, w_ref[...].astype(jnp.bfloat16),
                            (((1,), (1,)), ((), ())),
                            preferred_element_type=jnp.float32)
        return y + b_ref[...]

    def pack_i32(y):
        a = lax.bitcast_convert_type(
            y[:, :HALF].astype(jnp.bfloat16), jnp.uint16).astype(jnp.uint32)
        b = lax.bitcast_convert_type(
            y[:, HALF:].astype(jnp.bfloat16), jnp.uint16).astype(jnp.uint32)
        return lax.bitcast_convert_type(a | (b << 16), jnp.int32)

    q0_ref[...] = pack_i32(proj(wq_ref, bq_ref))
    k0_ref[...] = pack_i32(proj(wk_ref, bk_ref))
    v = proj(wv_ref, bv_ref)
    v0_ref[...] = v[:, :HALF]
    v1_ref[...] = v[:, HALF:]


def _proj(x, Wq, bq, Wk, bk, Wv, bv):
    RB = 2000
    w_spec = pl.BlockSpec((HIDDEN, HIDDEN), lambda i: (0, 0))
    b_spec = pl.BlockSpec((1, HIDDEN), lambda i: (0, 0))
    half_spec = pl.BlockSpec((RB, HALF), lambda i: (i, 0))
    return pl.pallas_call(
        _proj_body,
        grid=(N // RB,),
        in_specs=[pl.BlockSpec((RB, HIDDEN), lambda i: (i, 0)),
                  w_spec, b_spec, w_spec, b_spec, w_spec, b_spec],
        out_specs=[half_spec] * 4,
        out_shape=[jax.ShapeDtypeStruct((N, HALF), jnp.int32)] * 2
        + [jax.ShapeDtypeStruct((N, HALF), jnp.float32)] * 2,
    )(x, Wq, bq.reshape(1, HIDDEN), Wk, bk.reshape(1, HIDDEN),
      Wv, bv.reshape(1, HIDDEN))


def _logits_sc(qpack, kpack, src, tgt):
    mesh = plsc.VectorSubcoreMesh(core_axis_name="c", subcore_axis_name="s")

    @functools.partial(
        pl.kernel, mesh=mesh,
        compiler_params=pltpu.CompilerParams(needs_layout_passes=False),
        out_type=tuple(jax.ShapeDtypeStruct((E,), jnp.float32)
                       for _ in range(HEADS)),
        scratch_types=[
            pltpu.VMEM((SPAD,), jnp.int32),
            pltpu.VMEM((SPAD,), jnp.int32),
            pltpu.VMEM((EBL, HALF), jnp.int32),
            pltpu.VMEM((EBL, HALF), jnp.int32),
            pltpu.VMEM((EBL, HALF), jnp.int32),
            pltpu.VMEM((EBL, HALF), jnp.int32),
            pltpu.VMEM((SPAD,), jnp.float32),
            pltpu.VMEM((SPAD,), jnp.float32),
            pltpu.VMEM((SPAD,), jnp.float32),
            pltpu.VMEM((SPAD,), jnp.float32),
            pltpu.SemaphoreType.DMA,
            pltpu.SemaphoreType.DMA,
        ],
    )
    def logits_kernel(qh, kh, srch, tgth, o0, o1, o2, o3,
                      srcv, tgtv, qA, kA, qB, kB,
                      l0v, l1v, l2v, l3v, semA, semB):
        sid = lax.axis_index("s")
        cid = lax.axis_index("c")
        wid = cid * NSUB + sid
        base0 = wid * EPW2
        lvs = (l0v, l1v, l2v, l3v)

        pltpu.sync_copy(srch.at[pl.ds(base0, EPW2)], srcv.at[pl.ds(0, EPW2)])
        pltpu.sync_copy(tgth.at[pl.ds(base0, EPW2)], tgtv.at[pl.ds(0, EPW2)])
        zero16 = jnp.zeros((LANES,), jnp.int32)
        for off in list(range(EPW2, SPAD - 16, 16)) + [SPAD - 16]:
            srcv[pl.ds(off, LANES)] = zero16
            tgtv[pl.ds(off, LANES)] = zero16
        lane = lax.iota(jnp.int32, LANES)

        def issue(b, qb, kb, sem):
            off = b * EBL
            pltpu.async_copy(qh.at[tgtv.at[pl.ds(off, EBL)]], qb, sem)
            pltpu.async_copy(kh.at[srcv.at[pl.ds(off, EBL)]], kb, sem)

        def drain(qb, kb, sem):
            pltpu.make_async_copy(qh.at[pl.ds(0, EBL)], qb, sem).wait()
            pltpu.make_async_copy(kh.at[pl.ds(0, EBL)], kb, sem).wait()

        def compute(b, qb, kb):
            off = b * EBL

            def edots(e):
                acc = [None] * HEADS
                for j in range(8):
                    qw = plsc.bitcast(qb[e, pl.ds(16 * j, 16)], jnp.bfloat16)
                    kw = plsc.bitcast(kb[e, pl.ds(16 * j, 16)], jnp.bfloat16)
                    pw = plsc.unpack(qw * kw,
                                     format=plsc.PackFormat.INTERLEAVED)
                    lo = j // 4
                    acc[lo] = pw[0] if acc[lo] is None else acc[lo] + pw[0]
                    acc[2 + lo] = (pw[1] if acc[2 + lo] is None
                                   else acc[2 + lo] + pw[1])
                return [jnp.sum(a) for a in acc]

            def grp(g, c2):
                lvecs = [jnp.zeros((LANES,), jnp.float32) for _ in range(HEADS)]
                for i in range(LANES):
                    e = g * LANES + i
                    dots = edots(e)
                    for h in range(HEADS):
                        lvecs[h] = jnp.where(lane == i, dots[h], lvecs[h])
                for h in range(HEADS):
                    lvs[h][pl.ds(off + g * LANES, LANES)] = lvecs[h]
                return c2

            lax.fori_loop(0, GRL, grp, 0)

        issue(0, qA, kA, semA)

        def pair(t, carry):
            b0 = 2 * t
            issue(b0 + 1, qB, kB, semB)
            drain(qA, kA, semA)
            compute(b0, qA, kA)
            issue(b0 + 2, qA, kA, semA)
            drain(qB, kB, semB)
            compute(b0 + 1, qB, kB)
            return carry

        lax.fori_loop(0, (NBP2 - 1) // 2, pair, 0)
        drain(qA, kA, semA)
        compute(NBP2 - 1, qA, kA)
        for h, oh in enumerate((o0, o1, o2, o3)):
            pltpu.sync_copy(lvs[h].at[pl.ds(0, EPW2)],
                            oh.at[pl.ds(base0, EPW2)])

    return logits_kernel(qpack, kpack, src, tgt)


ER = E // 128


def _softmax_body(r0, r1, r2, r3, d_ref, g_ref, tau_ref, wd_ref, wg_ref,
                  w0, w1, w2, w3):
    tau_c = jnp.clip(tau_ref[0], 0.5, 5.0)
    scale = 1.0 / (8.0 * tau_c)
    bias = wd_ref[0] * d_ref[...] + wg_ref[0] * g_ref[...]
    for rr, ww in ((r0, w0), (r1, w1), (r2, w2), (r3, w3)):
        logit = rr[...] * scale + bias
        m = jnp.max(logit)
        p = jnp.exp(logit - m)
        s = jnp.sum(p)
        ww[...] = p / s


def _softmax(rs, dist, galign, tau, w_dist, w_galign):
    s_spec = pl.BlockSpec(memory_space=pltpu.SMEM)
    e_spec = pl.BlockSpec((ER, 128), lambda: (0, 0))
    return pl.pallas_call(
        _softmax_body,
        in_specs=[e_spec] * 6 + [s_spec, s_spec, s_spec],
        out_specs=[e_spec] * 4,
        out_shape=[jax.ShapeDtypeStruct((ER, 128), jnp.float32)] * 4,
    )(*[r.reshape(ER, 128) for r in rs],
      dist.reshape(ER, 128), galign.reshape(ER, 128),
      tau, w_dist, w_galign)


def _scatter_sc(v0, v1, src, tgt, wa, wb, wc, wd):
    mesh = plsc.VectorSubcoreMesh(core_axis_name="c", subcore_axis_name="s")

    @functools.partial(
        pl.kernel, mesh=mesh,
        compiler_params=pltpu.CompilerParams(needs_layout_passes=False),
        out_type=(jax.ShapeDtypeStruct((NPAD, HALF), jnp.float32),
                  jax.ShapeDtypeStruct((NPAD, HALF), jnp.float32)),
        scratch_types=[
            pltpu.VMEM((EBP,), jnp.int32),
            pltpu.VMEM((EBP,), jnp.int32),
            pltpu.VMEM((EBP,), jnp.int32),
            pltpu.VMEM((EBP,), jnp.int32),
            pltpu.VMEM((EBP,), jnp.int32),
            pltpu.VMEM((EBP,), jnp.int32),
            pltpu.VMEM((EBP, HALF), jnp.float32),
            pltpu.VMEM((EBP, HALF), jnp.float32),
            pltpu.VMEM((EBP, HALF), jnp.float32),
            pltpu.VMEM((EBP,), jnp.float32),
            pltpu.VMEM((EBP,), jnp.float32),
            pltpu.VMEM((EBP,), jnp.float32),
            pltpu.VMEM((EBP,), jnp.float32),
            pltpu.VMEM((EBP,), jnp.float32),
            pltpu.VMEM((EBP,), jnp.float32),
            pltpu.VMEM_SHARED((NPAD, HALF), jnp.float32),
            pltpu.SemaphoreType.DMA,
            pltpu.SemaphoreType.DMA,
            pltpu.SemaphoreType.DMA,
            pltpu.SemaphoreType.DMA,
            pltpu.SemaphoreType.DMA,
            pltpu.SemaphoreType.DMA,
            pltpu.SemaphoreType.DMA,
            pltpu.SemaphoreType.DMA,
            pltpu.SemaphoreType.DMA,
        ],
    )
    def scatter_kernel(v0h, v1h, srch, tgth, wah, wbh, wch, wdh, out0, out1,
                       iA, iB, iC, tA, tB, tC, vA, vB, vC,
                       w0A, w1A, w0B, w1B, w0C, w1C, acc,
                       gA, gB, gC, sA, sB, sC, siA, siB, siC):
        sid = lax.axis_index("s")
        cid = lax.axis_index("c")
        base0 = sid * EPW
        bufs = ((vA, tA, w0A, w1A, gA, sA),
                (vB, tB, w0B, w1B, gB, sB),
                (vC, tC, w0C, w1C, gC, sC))
        ibufs = ((iA, siA), (iB, siB), (iC, siC))

        def zrow(r, c2):
            for j in range(HALF // LANES):
                vA[r, pl.ds(16 * j, LANES)] = jnp.zeros((LANES,), jnp.float32)
            return c2

        lax.fori_loop(0, EBP, zrow, 0)
        for t in range(RPS // EBP):
            pltpu.sync_copy(vA, acc.at[pl.ds(sid * RPS + t * EBP, EBP)])
        plsc.subcore_barrier()

        def scatter_phase(vh, w0h, w1h):
            def issue_i(b, ibuf):
                ib, si = ibuf
                pltpu.async_copy(srch.at[pl.ds(base0 + b * EBP, EBP)], ib, si)

            def drain_i(ibuf):
                ib, si = ibuf
                pltpu.make_async_copy(srch.at[pl.ds(0, EBP)], ib, si).wait()

            def issue_g(b, buf, ibuf):
                vb, tb, w0b, w1b, g, _ = buf
                base = base0 + b * EBP
                pltpu.async_copy(vh.at[ibuf[0]], vb, g)
                pltpu.async_copy(w0h.at[pl.ds(base, EBP)], w0b, g)
                pltpu.async_copy(w1h.at[pl.ds(base, EBP)], w1b, g)
                pltpu.async_copy(tgth.at[pl.ds(base, EBP)], tb, g)

            def drain_g(buf):
                vb, tb, w0b, w1b, g, _ = buf
                pltpu.make_async_copy(vh.at[pl.ds(0, EBP)], vb, g).wait()
                pltpu.make_async_copy(wah.at[pl.ds(0, EBP)], w0b, g).wait()
                pltpu.make_async_copy(wah.at[pl.ds(0, EBP)], w1b, g).wait()
                pltpu.make_async_copy(srch.at[pl.ds(0, EBP)], tb, g).wait()

            def scale(buf):
                vb, _, w0b, w1b, _, _ = buf

                def grp(g, c2):
                    w0vec = w0b[pl.ds(g * LANES, LANES)]
                    w1vec = w1b[pl.ds(g * LANES, LANES)]
                    for i in range(LANES):
                        e = g * LANES + i
                        w0 = w0vec[i]
                        w1 = w1vec[i]
                        for j in range(4):
                            vb[e, pl.ds(16 * j, LANES)] = vb[e, pl.ds(16 * j, LANES)] * w0
                        for j in range(4, 8):
                            vb[e, pl.ds(16 * j, LANES)] = vb[e, pl.ds(16 * j, LANES)] * w1
                    return c2

                lax.fori_loop(0, GRP, grp, 0)

            def issue_s(buf):
                vb, tb, _, _, _, s = buf
                pltpu.async_copy(vb, acc.at[tb], s, add=True)

            def drain_s(buf):
                vb, _, _, _, _, s = buf
                pltpu.make_async_copy(vb, acc.at[pl.ds(0, EBP)], s).wait()

            pltpu.sync_copy(srch.at[pl.ds(base0, EBP)], iA)
            issue_i(1, ibufs[1])
            issue_g(0, bufs[0], ibufs[0])

            def triple(t, carry):
                b0 = 3 * t
                for r in range(3):
                    b = b0 + r
                    W = bufs[r]
                    G = bufs[(r + 1) % 3]
                    if r < 2:
                        @pl.when(t > 0)
                        def _():
                            drain_s(G)
                    else:
                        drain_s(G)
                    drain_i(ibufs[(r + 1) % 3])
                    issue_g(b + 1, G, ibufs[(r + 1) % 3])
                    drain_g(W)
                    issue_i(b + 2, ibufs[(r + 2) % 3])
                    scale(W)
                    issue_s(W)
                return carry

            lax.fori_loop(0, NBP // 3, triple, 0)
            drain_s(bufs[1])
            drain_i(ibufs[1])
            issue_g(NBP - 1, bufs[1], ibufs[1])
            drain_g(bufs[0])
            scale(bufs[0])
            issue_s(bufs[0])
            drain_s(bufs[2])
            drain_g(bufs[1])
            scale(bufs[1])
            issue_s(bufs[1])
            drain_s(bufs[0])
            drain_s(bufs[1])

        @pl.when(cid == 0)
        def _():
            scatter_phase(v0h, wah, wbh)

        @pl.when(cid == 1)
        def _():
            scatter_phase(v1h, wch, wdh)

        plsc.subcore_barrier()

        def writeback(outh):
            r = sid * RPS
            pltpu.sync_copy(acc.at[pl.ds(r, RPS)], outh.at[pl.ds(r, RPS)])

        @pl.when(cid == 0)
        def _():
            writeback(out0)

        @pl.when(cid == 1)
        def _():
            writeback(out1)

    return scatter_kernel(v0, v1, src, tgt, wa, wb, wc, wd)


def _out_body(o0_ref, o1_ref, wo_ref, bo_ref, f_ref):
    w = wo_ref[...].astype(jnp.bfloat16)
    f = lax.dot_general(o0_ref[...].astype(jnp.bfloat16), w[:, :HALF],
                        (((1,), (1,)), ((), ())),
                        preferred_element_type=jnp.float32)
    f = f + lax.dot_general(o1_ref[...].astype(jnp.bfloat16), w[:, HALF:],
                            (((1,), (1,)), ((), ())),
                            preferred_element_type=jnp.float32)
    f_ref[...] = f + bo_ref[...]


def _outproj(o0, o1, Wo, bo):
    RB = 2000
    return pl.pallas_call(
        _out_body,
        grid=(N // RB,),
        in_specs=[pl.BlockSpec((RB, HALF), lambda i: (i, 0)),
                  pl.BlockSpec((RB, HALF), lambda i: (i, 0)),
                  pl.BlockSpec((HIDDEN, HIDDEN), lambda i: (0, 0)),
                  pl.BlockSpec((1, HIDDEN), lambda i: (0, 0))],
        out_specs=pl.BlockSpec((RB, HIDDEN), lambda i: (i, 0)),
        out_shape=jax.ShapeDtypeStruct((N, HIDDEN), jnp.float32),
    )(o0, o1, Wo, bo.reshape(1, HIDDEN))


def kernel(x, edge_index, edge_attr, Wq, bq, Wk, bk, Wv, bv, Wo, bo,
           w_dist, w_galign, tau):
    src = edge_index[0].astype(jnp.int32)
    tgt = edge_index[1].astype(jnp.int32)
    dist = edge_attr[:, 2]
    galign = edge_attr[:, 3]

    qi, ki, v0, v1 = _proj(x, Wq, bq, Wk, bk, Wv, bv)
    rs = _logits_sc(qi, ki, src, tgt)
    ws = _softmax(rs, dist, galign, tau, w_dist, w_galign)
    o0p, o1p = _scatter_sc(v0, v1, src, tgt,
                           *[w.reshape(E) for w in ws])
    return _outproj(o0p, o1p, Wo, bo)
